# baseline (speedup 1.0000x reference)
import jax
import jax.numpy as jnp
from jax import lax
from jax.experimental import pallas as pl
from jax.experimental.pallas import tpu as pltpu
from jax.experimental.pallas import tpu_sc as plsc

_NC = 2
_NS = 16
_NW = _NC * _NS
_CH = 128


def _largest_div_le(n, cap):
    for cand in range(min(n, cap), 0, -1):
        if n % cand == 0:
            return cand
    return 1


def _acc_rows(n):
    nps = -(-n // (_NS * 8)) * 8
    return nps * _NS, nps


def _sc_edge_aggregate(table, src, dst):
    n, d = table.shape
    e = src.shape[0]
    nchunk = e // _CH
    steps = nchunk // _NW
    assert nchunk == steps * _NW
    npad, nps = _acc_rows(n)
    zr = _largest_div_le(nps, 128)
    nz = nps // zr

    scratch = [
        pltpu.VMEM((_CH,), jnp.int32),
        pltpu.VMEM((_CH,), jnp.int32),
        pltpu.VMEM((_CH, d), jnp.float32),
        pltpu.VMEM((zr, d), jnp.float32),
        pltpu.VMEM_SHARED((npad, d), jnp.float32),
        pltpu.SemaphoreType.DMA,
    ]

    def body(x_hbm, src_hbm, dst_hbm, out_hbm, src_v, dst_v, rows_v,
             zrow_v, acc_sh, sem):
        cid = lax.axis_index("c")
        sid = lax.axis_index("s")
        w = sid * _NC + cid

        zero16 = jnp.zeros((16,), jnp.float32)

        @pl.loop(0, zr)
        def _(r):
            for j in range(d // 16):
                zrow_v[r, pl.ds(16 * j, 16)] = zero16

        base = sid * nps

        @pl.loop(0, nz)
        def _(z):
            pltpu.sync_copy(zrow_v, acc_sh.at[pl.ds(base + z * zr, zr)])

        plsc.subcore_barrier()

        @pl.loop(0, steps)
        def _(i):
            off = (w + i * _NW) * _CH
            pltpu.sync_copy(src_hbm.at[pl.ds(off, _CH)], src_v)
            pltpu.sync_copy(dst_hbm.at[pl.ds(off, _CH)], dst_v)
            pltpu.async_copy(x_hbm.at[src_v], rows_v, sem).wait()
            pltpu.sync_copy(rows_v, acc_sh.at[dst_v], add=True)

        plsc.subcore_barrier()
        obase = cid * npad + base
        pltpu.sync_copy(acc_sh.at[pl.ds(base, nps)],
                        out_hbm.at[pl.ds(obase, nps)])

    mesh = plsc.VectorSubcoreMesh(core_axis_name="c", subcore_axis_name="s")
    f = pl.kernel(body,
                  out_type=jax.ShapeDtypeStruct((_NC * npad, d), jnp.float32),
                  mesh=mesh, scratch_types=scratch)
    return f(table, src, dst), npad


def _sc_degree_count(dst, n):
    e = dst.shape[0]
    nchunk = e // _CH
    steps = nchunk // _NW
    assert nchunk == steps * _NW and steps % 4 == 0 and steps >= 8
    npad, nps = _acc_rows(n)
    zr = _largest_div_le(nps, 128)
    nz = nps // zr

    scratch = [
        pltpu.VMEM((4, _CH), jnp.int32),
        pltpu.VMEM((zr, 128), jnp.float32),
        pltpu.VMEM((_CH, 128), jnp.float32),
        pltpu.VMEM_SHARED((npad, 128), jnp.float32),
    ] + [pltpu.SemaphoreType.DMA] * 6

    def body(dst_hbm, cnt_hbm, dstb, zc_v, ones_v, cacc_sh,
             si0, si1, si2, si3, ss0, ss1):
        cid = lax.axis_index("c")
        sid = lax.axis_index("s")
        w = sid * _NC + cid
        sem_i = (si0, si1, si2, si3)
        sem_s = (ss0, ss1)

        zero16 = jnp.zeros((16,), jnp.float32)
        one16 = jnp.ones((16,), jnp.float32)

        @pl.loop(0, zr)
        def _(r):
            for j in range(8):
                zc_v[r, pl.ds(16 * j, 16)] = zero16

        @pl.loop(0, _CH)
        def _(r):
            for j in range(8):
                ones_v[r, pl.ds(16 * j, 16)] = one16

        base = sid * nps

        @pl.loop(0, nz)
        def _(z):
            pltpu.sync_copy(zc_v, cacc_sh.at[pl.ds(base + z * zr, zr)])

        plsc.subcore_barrier()

        def issue_i(step, bi):
            off = (w + step * _NW) * _CH
            pltpu.async_copy(dst_hbm.at[pl.ds(off, _CH)], dstb.at[bi],
                             sem_i[bi])

        def wait_i(bi):
            pltpu.make_async_copy(dst_hbm.at[pl.ds(0, _CH)], dstb.at[bi],
                                  sem_i[bi]).wait()

        def issue_s(bi, b):
            pltpu.async_copy(ones_v, cacc_sh.at[dstb.at[bi]], sem_s[b],
                             add=True)

        def wait_s(b):
            pltpu.make_async_copy(cnt_hbm.at[pl.ds(0, _CH)], ones_v,
                                  sem_s[b]).wait()

        issue_i(0, 0)
        issue_i(1, 1)
        wait_i(0)
        issue_s(0, 0)
        issue_i(2, 2)
        wait_i(1)
        issue_s(1, 1)
        issue_i(3, 3)

        @pl.loop(0, (steps - 4) // 4)
        def _(o):
            for u in range(4):
                step = 2 + o * 4 + u
                b = u % 2
                bi = (2 + u) % 4
                wait_i(bi)
                issue_s(bi, b)
                wait_s(b)
                issue_i(step + 2, (bi + 2) % 4)

        wait_i(2)
        issue_s(2, 0)
        wait_s(0)
        wait_i(3)
        issue_s(3, 1)
        wait_s(1)
        wait_s(0)
        wait_s(1)

        plsc.subcore_barrier()
        obase = cid * npad + base
        pltpu.sync_copy(cacc_sh.at[pl.ds(base, nps)],
                        cnt_hbm.at[pl.ds(obase, nps)])

    mesh = plsc.VectorSubcoreMesh(core_axis_name="c", subcore_axis_name="s")
    f = pl.kernel(body,
                  out_type=jax.ShapeDtypeStruct((_NC * npad, 128),
                                                jnp.float32),
                  mesh=mesh, scratch_types=scratch)
    return f(dst)


def _sc_pool(h, batch, g):
    n, d = h.shape
    full = n // _CH
    assert full * _CH == n
    iters = -(-full // _NW)
    ga = g + 8
    assert g % 8 == 0 and ga // 8 <= _NS

    scratch = [
        pltpu.VMEM((_CH,), jnp.int32),
        pltpu.VMEM((_CH, d), jnp.float32),
        pltpu.VMEM((_CH, 128), jnp.float32),
        pltpu.VMEM((8, d), jnp.float32),
        pltpu.VMEM((8, 128), jnp.float32),
        pltpu.VMEM_SHARED((ga, d), jnp.float32),
        pltpu.VMEM_SHARED((ga, 128), jnp.float32),
        pltpu.SemaphoreType.DMA,
    ]

    def body(h_hbm, b_hbm, out_hbm, cnt_hbm, bidx_v, rows_v, ones_v,
             zrow_v, zc_v, acc_sh, cacc_sh, sem):
        cid = lax.axis_index("c")
        sid = lax.axis_index("s")
        w = sid * _NC + cid

        zero16 = jnp.zeros((16,), jnp.float32)
        one16 = jnp.ones((16,), jnp.float32)

        @pl.loop(0, 8)
        def _(r):
            for j in range(d // 16):
                zrow_v[r, pl.ds(16 * j, 16)] = zero16
            for j in range(8):
                zc_v[r, pl.ds(16 * j, 16)] = zero16

        @pl.loop(0, _CH)
        def _(r):
            for j in range(8):
                ones_v[r, pl.ds(16 * j, 16)] = one16

        base = sid * 8

        @pl.when(sid < ga // 8)
        def _():
            pltpu.sync_copy(zrow_v, acc_sh.at[pl.ds(base, 8)])
            pltpu.sync_copy(zc_v, cacc_sh.at[pl.ds(base, 8)])

        plsc.subcore_barrier()

        @pl.loop(0, iters)
        def _(i):
            c = w + i * _NW

            @pl.when(c < full)
            def _():
                off = c * _CH
                pltpu.sync_copy(b_hbm.at[pl.ds(off, _CH)], bidx_v)
                pltpu.sync_copy(h_hbm.at[pl.ds(off, _CH)], rows_v)
                pltpu.sync_copy(rows_v, acc_sh.at[bidx_v], add=True)
                pltpu.sync_copy(ones_v, cacc_sh.at[bidx_v], add=True)

        plsc.subcore_barrier()
        obase = cid * g + base

        @pl.when(sid < g // 8)
        def _():
            pltpu.sync_copy(acc_sh.at[pl.ds(base, 8)],
                            out_hbm.at[pl.ds(obase, 8)])
            pltpu.sync_copy(cacc_sh.at[pl.ds(base, 8)],
                            cnt_hbm.at[pl.ds(obase, 8)])

    mesh = plsc.VectorSubcoreMesh(core_axis_name="c", subcore_axis_name="s")
    f = pl.kernel(body,
                  out_type=(jax.ShapeDtypeStruct((_NC * g, d), jnp.float32),
                            jax.ShapeDtypeStruct((_NC * g, 128),
                                                 jnp.float32)),
                  mesh=mesh, scratch_types=scratch)
    return f(h, batch)


def _tc_sage_linear(parts, cnts, x, wl, bias, wr, npad):
    d = x.shape[1]
    br = _largest_div_le(npad, 1024)
    while br % 8 != 0:
        br //= 2
    nb = npad // br
    off = nb

    def body(p0, p1, c0, c1, x_ref, wl_ref, wr_ref, b_ref, o_ref):
        s = p0[...] + p1[...]
        c = c0[...][:, 0:1] + c1[...][:, 0:1]
        agg = s / jnp.maximum(c, 1.0)
        h = (lax.dot_general(agg, wl_ref[...], (((1,), (1,)), ((), ())),
                             preferred_element_type=jnp.float32)
             + lax.dot_general(x_ref[...], wr_ref[...],
                               (((1,), (1,)), ((), ())),
                               preferred_element_type=jnp.float32)
             + b_ref[...])
        o_ref[...] = jnp.maximum(h, 0.0)

    row_spec = pl.BlockSpec((br, d), lambda i: (i, 0))
    return pl.pallas_call(
        body,
        grid=(nb,),
        in_specs=[
            row_spec,
            pl.BlockSpec((br, d), lambda i: (i + off, 0)),
            pl.BlockSpec((br, 128), lambda i: (i, 0)),
            pl.BlockSpec((br, 128), lambda i: (i + off, 0)),
            row_spec,
            pl.BlockSpec((d, d), lambda i: (0, 0)),
            pl.BlockSpec((d, d), lambda i: (0, 0)),
            pl.BlockSpec((1, d), lambda i: (0, 0)),
        ],
        out_specs=row_spec,
        out_shape=jax.ShapeDtypeStruct((npad, d), jnp.float32),
    )(parts, parts, cnts, cnts, x, wl, wr, bias.reshape(1, d))


def _tc_head(pparts, pcnts, wc, bias, g, d):
    def body(p0, p1, c0, c1, w_ref, b_ref, o_ref):
        p = p0[...] + p1[...]
        c = c0[...][:, 0:1] + c1[...][:, 0:1]
        pooled = p / jnp.maximum(c, 1.0)
        o_ref[...] = (lax.dot_general(pooled, w_ref[...],
                                      (((1,), (1,)), ((), ())),
                                      preferred_element_type=jnp.float32)
                      + b_ref[...])

    return pl.pallas_call(
        body,
        grid=(1,),
        in_specs=[
            pl.BlockSpec((g, d), lambda i: (0, 0)),
            pl.BlockSpec((g, d), lambda i: (1, 0)),
            pl.BlockSpec((g, 128), lambda i: (0, 0)),
            pl.BlockSpec((g, 128), lambda i: (1, 0)),
            pl.BlockSpec((d, d), lambda i: (0, 0)),
            pl.BlockSpec((1, d), lambda i: (0, 0)),
        ],
        out_specs=pl.BlockSpec((g, d), lambda i: (0, 0)),
        out_shape=jax.ShapeDtypeStruct((g, d), jnp.float32),
    )(pparts, pparts, pcnts, pcnts, wc, bias.reshape(1, d))


def kernel(x, edge_index, batch, W1l, b1l, W1r, W2l, b2l, W2r, Wc, bc):
    g = 64
    n, d = x.shape
    e = edge_index.shape[1]
    npad, _ = _acc_rows(n)
    assert npad > n

    steps = -(-e // (_NW * _CH))
    steps = max(-(-steps // 4) * 4, 8)
    epad = steps * _NW * _CH
    src = jnp.concatenate(
        [edge_index[0], jnp.zeros((epad - e,), jnp.int32)])
    dst = jnp.concatenate(
        [edge_index[1], jnp.full((epad - e,), n, jnp.int32)])
    parts1, npad1 = _sc_edge_aggregate(x, src, dst)
    cnts = _sc_degree_count(dst, n)
    h1 = _tc_sage_linear(parts1, cnts, x, W1l, b1l, W1r, npad1)
    parts2, npad2 = _sc_edge_aggregate(h1, src, dst)
    h2 = _tc_sage_linear(parts2, cnts, h1, W2l, b2l, W2r, npad2)

    batch_p = jnp.concatenate(
        [batch, jnp.full((h2.shape[0] - n,), g, jnp.int32)])
    pparts, pcnts = _sc_pool(h2, batch_p, g)
    return _tc_head(pparts, pcnts, Wc, bc, g, d)


# speedup vs baseline: 1.0023x; 1.0023x over previous
import jax
import jax.numpy as jnp
from jax import lax
from jax.experimental import pallas as pl
from jax.experimental.pallas import tpu as pltpu
from jax.experimental.pallas import tpu_sc as plsc

_NC = 2
_NS = 16
_NW = _NC * _NS
_CH = 128


def _largest_div_le(n, cap):
    for cand in range(min(n, cap), 0, -1):
        if n % cand == 0:
            return cand
    return 1


def _acc_rows(n):
    nps = -(-n // (_NS * 8)) * 8
    return nps * _NS, nps


def _sc_edge_aggregate(table, src, dst):
    n, d = table.shape
    e = src.shape[0]
    nchunk = e // _CH
    steps = nchunk // _NW
    assert nchunk == steps * _NW
    npad, nps = _acc_rows(n)
    zr = _largest_div_le(nps, 128)
    nz = nps // zr

    scratch = [
        pltpu.VMEM((_CH,), jnp.int32),
        pltpu.VMEM((_CH,), jnp.int32),
        pltpu.VMEM((_CH, d), jnp.float32),
        pltpu.VMEM((zr, d), jnp.float32),
        pltpu.VMEM_SHARED((npad, d), jnp.float32),
        pltpu.SemaphoreType.DMA,
    ]

    def body(x_hbm, src_hbm, dst_hbm, out_hbm, src_v, dst_v, rows_v,
             zrow_v, acc_sh, sem):
        cid = lax.axis_index("c")
        sid = lax.axis_index("s")
        w = sid * _NC + cid

        zero16 = jnp.zeros((16,), jnp.float32)

        @pl.loop(0, zr)
        def _(r):
            for j in range(d // 16):
                zrow_v[r, pl.ds(16 * j, 16)] = zero16

        base = sid * nps

        @pl.loop(0, nz)
        def _(z):
            pltpu.sync_copy(zrow_v, acc_sh.at[pl.ds(base + z * zr, zr)])

        plsc.subcore_barrier()

        @pl.loop(0, steps)
        def _(i):
            off = (w + i * _NW) * _CH
            pltpu.sync_copy(src_hbm.at[pl.ds(off, _CH)], src_v)
            pltpu.sync_copy(dst_hbm.at[pl.ds(off, _CH)], dst_v)
            pltpu.async_copy(x_hbm.at[src_v], rows_v, sem).wait()
            pltpu.sync_copy(rows_v, acc_sh.at[dst_v], add=True)

        plsc.subcore_barrier()
        obase = cid * npad + base
        pltpu.sync_copy(acc_sh.at[pl.ds(base, nps)],
                        out_hbm.at[pl.ds(obase, nps)])

    mesh = plsc.VectorSubcoreMesh(core_axis_name="c", subcore_axis_name="s")
    f = pl.kernel(body,
                  out_type=jax.ShapeDtypeStruct((_NC * npad, d), jnp.float32),
                  mesh=mesh, scratch_types=scratch)
    return f(table, src, dst), npad


def _sc_degree_count(dst, n):
    e = dst.shape[0]
    nchunk = e // _CH
    steps = nchunk // _NW
    assert nchunk == steps * _NW and steps % 4 == 0 and steps >= 8
    npad, nps = _acc_rows(n)
    zr = _largest_div_le(nps, 128)
    nz = nps // zr

    scratch = [
        pltpu.VMEM((4, _CH), jnp.int32),
        pltpu.VMEM((zr, 128), jnp.float32),
        pltpu.VMEM((_CH, 128), jnp.float32),
        pltpu.VMEM_SHARED((npad, 128), jnp.float32),
    ] + [pltpu.SemaphoreType.DMA] * 6

    def body(dst_hbm, cnt_hbm, dstb, zc_v, ones_v, cacc_sh,
             si0, si1, si2, si3, ss0, ss1):
        cid = lax.axis_index("c")
        sid = lax.axis_index("s")
        w = sid * _NC + cid
        sem_i = (si0, si1, si2, si3)
        sem_s = (ss0, ss1)

        zero16 = jnp.zeros((16,), jnp.float32)
        one16 = jnp.ones((16,), jnp.float32)

        @pl.loop(0, zr)
        def _(r):
            for j in range(8):
                zc_v[r, pl.ds(16 * j, 16)] = zero16

        @pl.loop(0, _CH)
        def _(r):
            for j in range(8):
                ones_v[r, pl.ds(16 * j, 16)] = one16

        base = sid * nps

        @pl.loop(0, nz)
        def _(z):
            pltpu.sync_copy(zc_v, cacc_sh.at[pl.ds(base + z * zr, zr)])

        plsc.subcore_barrier()

        def issue_i(step, bi):
            off = (w + step * _NW) * _CH
            pltpu.async_copy(dst_hbm.at[pl.ds(off, _CH)], dstb.at[bi],
                             sem_i[bi])

        def wait_i(bi):
            pltpu.make_async_copy(dst_hbm.at[pl.ds(0, _CH)], dstb.at[bi],
                                  sem_i[bi]).wait()

        def issue_s(bi, b):
            pltpu.async_copy(ones_v, cacc_sh.at[dstb.at[bi]], sem_s[b],
                             add=True)

        def wait_s(b):
            pltpu.make_async_copy(cnt_hbm.at[pl.ds(0, _CH)], ones_v,
                                  sem_s[b]).wait()

        issue_i(0, 0)
        issue_i(1, 1)
        wait_i(0)
        issue_s(0, 0)
        issue_i(2, 2)
        wait_i(1)
        issue_s(1, 1)
        issue_i(3, 3)

        @pl.loop(0, (steps - 4) // 4)
        def _(o):
            for u in range(4):
                step = 2 + o * 4 + u
                b = u % 2
                bi = (2 + u) % 4
                wait_i(bi)
                issue_s(bi, b)
                wait_s(b)
                issue_i(step + 2, (bi + 2) % 4)

        wait_i(2)
        issue_s(2, 0)
        wait_s(0)
        wait_i(3)
        issue_s(3, 1)
        wait_s(1)
        wait_s(0)
        wait_s(1)

        plsc.subcore_barrier()
        obase = cid * npad + base
        pltpu.sync_copy(cacc_sh.at[pl.ds(base, nps)],
                        cnt_hbm.at[pl.ds(obase, nps)])

    mesh = plsc.VectorSubcoreMesh(core_axis_name="c", subcore_axis_name="s")
    f = pl.kernel(body,
                  out_type=jax.ShapeDtypeStruct((_NC * npad, 128),
                                                jnp.float32),
                  mesh=mesh, scratch_types=scratch)
    return f(dst)


def _sc_pool(h, batch, g):
    n, d = h.shape
    full = n // _CH
    assert full * _CH == n
    iters = -(-full // _NW)
    ga = g + 8
    assert g % 8 == 0 and ga // 8 <= _NS

    scratch = [
        pltpu.VMEM((_CH,), jnp.int32),
        pltpu.VMEM((_CH, d), jnp.float32),
        pltpu.VMEM((_CH, 128), jnp.float32),
        pltpu.VMEM((8, d), jnp.float32),
        pltpu.VMEM((8, 128), jnp.float32),
        pltpu.VMEM_SHARED((ga, d), jnp.float32),
        pltpu.VMEM_SHARED((ga, 128), jnp.float32),
        pltpu.SemaphoreType.DMA,
    ]

    def body(h_hbm, b_hbm, out_hbm, cnt_hbm, bidx_v, rows_v, ones_v,
             zrow_v, zc_v, acc_sh, cacc_sh, sem):
        cid = lax.axis_index("c")
        sid = lax.axis_index("s")
        w = sid * _NC + cid

        zero16 = jnp.zeros((16,), jnp.float32)
        one16 = jnp.ones((16,), jnp.float32)

        @pl.loop(0, 8)
        def _(r):
            for j in range(d // 16):
                zrow_v[r, pl.ds(16 * j, 16)] = zero16
            for j in range(8):
                zc_v[r, pl.ds(16 * j, 16)] = zero16

        @pl.loop(0, _CH)
        def _(r):
            for j in range(8):
                ones_v[r, pl.ds(16 * j, 16)] = one16

        base = sid * 8

        @pl.when(sid < ga // 8)
        def _():
            pltpu.sync_copy(zrow_v, acc_sh.at[pl.ds(base, 8)])
            pltpu.sync_copy(zc_v, cacc_sh.at[pl.ds(base, 8)])

        plsc.subcore_barrier()

        @pl.loop(0, iters)
        def _(i):
            c = w + i * _NW

            @pl.when(c < full)
            def _():
                off = c * _CH
                pltpu.sync_copy(b_hbm.at[pl.ds(off, _CH)], bidx_v)
                pltpu.sync_copy(h_hbm.at[pl.ds(off, _CH)], rows_v)
                pltpu.sync_copy(rows_v, acc_sh.at[bidx_v], add=True)
                pltpu.sync_copy(ones_v, cacc_sh.at[bidx_v], add=True)

        plsc.subcore_barrier()
        obase = cid * g + base

        @pl.when(sid < g // 8)
        def _():
            pltpu.sync_copy(acc_sh.at[pl.ds(base, 8)],
                            out_hbm.at[pl.ds(obase, 8)])
            pltpu.sync_copy(cacc_sh.at[pl.ds(base, 8)],
                            cnt_hbm.at[pl.ds(obase, 8)])

    mesh = plsc.VectorSubcoreMesh(core_axis_name="c", subcore_axis_name="s")
    f = pl.kernel(body,
                  out_type=(jax.ShapeDtypeStruct((_NC * g, d), jnp.float32),
                            jax.ShapeDtypeStruct((_NC * g, 128),
                                                 jnp.float32)),
                  mesh=mesh, scratch_types=scratch)
    return f(h, batch)


def _tc_sage_linear(parts, cnts, x, wl, bias, wr, npad):
    d = x.shape[1]
    br = _largest_div_le(npad, 1024)
    while br % 8 != 0:
        br //= 2
    nb = npad // br
    off = nb

    def body(p0, p1, c0, c1, x_ref, wl_ref, wr_ref, b_ref, o_ref):
        s = p0[...] + p1[...]
        c = c0[...][:, 0:1] + c1[...][:, 0:1]
        agg = s / jnp.maximum(c, 1.0)
        h = (lax.dot_general(agg, wl_ref[...], (((1,), (1,)), ((), ())),
                             preferred_element_type=jnp.float32)
             + lax.dot_general(x_ref[...], wr_ref[...],
                               (((1,), (1,)), ((), ())),
                               preferred_element_type=jnp.float32)
             + b_ref[...])
        o_ref[...] = jnp.maximum(h, 0.0)

    row_spec = pl.BlockSpec((br, d), lambda i: (i, 0))
    return pl.pallas_call(
        body,
        grid=(nb,),
        in_specs=[
            row_spec,
            pl.BlockSpec((br, d), lambda i: (i + off, 0)),
            pl.BlockSpec((br, 128), lambda i: (i, 0)),
            pl.BlockSpec((br, 128), lambda i: (i + off, 0)),
            row_spec,
            pl.BlockSpec((d, d), lambda i: (0, 0)),
            pl.BlockSpec((d, d), lambda i: (0, 0)),
            pl.BlockSpec((1, d), lambda i: (0, 0)),
        ],
        out_specs=row_spec,
        out_shape=jax.ShapeDtypeStruct((npad, d), jnp.float32),
    )(parts, parts, cnts, cnts, x, wl, wr, bias.reshape(1, d))


def _tc_head(pparts, pcnts, wc, bias, g, d):
    def body(p0, p1, c0, c1, w_ref, b_ref, o_ref):
        p = p0[...] + p1[...]
        c = c0[...][:, 0:1] + c1[...][:, 0:1]
        pooled = p / jnp.maximum(c, 1.0)
        o_ref[...] = (lax.dot_general(pooled, w_ref[...],
                                      (((1,), (1,)), ((), ())),
                                      preferred_element_type=jnp.float32)
                      + b_ref[...])

    return pl.pallas_call(
        body,
        grid=(1,),
        in_specs=[
            pl.BlockSpec((g, d), lambda i: (0, 0)),
            pl.BlockSpec((g, d), lambda i: (1, 0)),
            pl.BlockSpec((g, 128), lambda i: (0, 0)),
            pl.BlockSpec((g, 128), lambda i: (1, 0)),
            pl.BlockSpec((d, d), lambda i: (0, 0)),
            pl.BlockSpec((1, d), lambda i: (0, 0)),
        ],
        out_specs=pl.BlockSpec((g, d), lambda i: (0, 0)),
        out_shape=jax.ShapeDtypeStruct((g, d), jnp.float32),
    )(pparts, pparts, pcnts, pcnts, wc, bias.reshape(1, d))


def kernel(x, edge_index, batch, W1l, b1l, W1r, W2l, b2l, W2r, Wc, bc):
    g = 64
    n, d = x.shape
    e = edge_index.shape[1]
    npad, _ = _acc_rows(n)
    assert npad > n

    steps = -(-e // (_NW * _CH))
    steps = max(-(-steps // 4) * 4, 8)
    epad = steps * _NW * _CH
    src = jnp.concatenate(
        [edge_index[0], jnp.zeros((epad - e,), jnp.int32)])
    pad_dst = n + jnp.arange(epad - e, dtype=jnp.int32) % (npad - n)
    dst = jnp.concatenate([edge_index[1], pad_dst])
    parts1, npad1 = _sc_edge_aggregate(x, src, dst)
    cnts = _sc_degree_count(dst, n)
    h1 = _tc_sage_linear(parts1, cnts, x, W1l, b1l, W1r, npad1)
    parts2, npad2 = _sc_edge_aggregate(h1, src, dst)
    h2 = _tc_sage_linear(parts2, cnts, h1, W2l, b2l, W2r, npad2)

    batch_p = jnp.concatenate(
        [batch, jnp.full((h2.shape[0] - n,), g, jnp.int32)])
    pparts, pcnts = _sc_pool(h2, batch_p, g)
    return _tc_head(pparts, pcnts, Wc, bc, g, d)


# speedup vs baseline: 1.8572x; 1.8529x over previous
import jax
import jax.numpy as jnp
from jax import lax
from jax.experimental import pallas as pl
from jax.experimental.pallas import tpu as pltpu
from jax.experimental.pallas import tpu_sc as plsc

_NC = 2
_NS = 16
_NW = _NC * _NS
_CH = 128


def _largest_div_le(n, cap):
    for cand in range(min(n, cap), 0, -1):
        if n % cand == 0:
            return cand
    return 1


def _acc_rows(n):
    nps = -(-n // (_NS * 8)) * 8
    return nps * _NS, nps


def _sc_edge_aggregate(table, src, dst):
    n, d = table.shape
    e = src.shape[0]
    nchunk = e // _CH
    steps = nchunk // _NW
    assert nchunk == steps * _NW
    npad, nps = _acc_rows(n)
    zr = _largest_div_le(nps, 128)
    nz = nps // zr

    scratch = [
        pltpu.VMEM((_CH,), jnp.int32),
        pltpu.VMEM((_CH,), jnp.int32),
        pltpu.VMEM((_CH, d), jnp.float32),
        pltpu.VMEM((zr, d), jnp.float32),
        pltpu.VMEM_SHARED((npad, d), jnp.float32),
        pltpu.SemaphoreType.DMA,
    ]

    def body(x_hbm, src_hbm, dst_hbm, out_hbm, src_v, dst_v, rows_v,
             zrow_v, acc_sh, sem):
        cid = lax.axis_index("c")
        sid = lax.axis_index("s")
        w = sid * _NC + cid

        zero16 = jnp.zeros((16,), jnp.float32)

        @pl.loop(0, zr)
        def _(r):
            for j in range(d // 16):
                zrow_v[r, pl.ds(16 * j, 16)] = zero16

        base = sid * nps

        @pl.loop(0, nz)
        def _(z):
            pltpu.sync_copy(zrow_v, acc_sh.at[pl.ds(base + z * zr, zr)])

        plsc.subcore_barrier()

        @pl.loop(0, steps)
        def _(i):
            off = (w + i * _NW) * _CH
            pltpu.sync_copy(src_hbm.at[pl.ds(off, _CH)], src_v)
            pltpu.sync_copy(dst_hbm.at[pl.ds(off, _CH)], dst_v)
            pltpu.async_copy(x_hbm.at[src_v], rows_v, sem).wait()
            pltpu.sync_copy(rows_v, acc_sh.at[dst_v], add=True)

        plsc.subcore_barrier()
        obase = cid * npad + base
        pltpu.sync_copy(acc_sh.at[pl.ds(base, nps)],
                        out_hbm.at[pl.ds(obase, nps)])

    mesh = plsc.VectorSubcoreMesh(core_axis_name="c", subcore_axis_name="s")
    f = pl.kernel(body,
                  out_type=jax.ShapeDtypeStruct((_NC * npad, d), jnp.float32),
                  mesh=mesh, scratch_types=scratch)
    return f(table, src, dst), npad


def _sc_degree_count(dst, n):
    e = dst.shape[0]
    nchunk = e // _CH
    steps = nchunk // _NW
    assert nchunk == steps * _NW and steps % 4 == 0 and steps >= 8
    npad, nps = _acc_rows(n)
    zr = _largest_div_le(nps, 128)
    nz = nps // zr

    scratch = [
        pltpu.VMEM((4, _CH), jnp.int32),
        pltpu.VMEM((zr, 128), jnp.float32),
        pltpu.VMEM((_CH, 128), jnp.float32),
        pltpu.VMEM_SHARED((npad, 128), jnp.float32),
    ] + [pltpu.SemaphoreType.DMA] * 6

    def body(dst_hbm, cnt_hbm, dstb, zc_v, ones_v, cacc_sh,
             si0, si1, si2, si3, ss0, ss1):
        cid = lax.axis_index("c")
        sid = lax.axis_index("s")
        w = sid * _NC + cid
        sem_i = (si0, si1, si2, si3)
        sem_s = (ss0, ss1)

        zero16 = jnp.zeros((16,), jnp.float32)
        one16 = jnp.ones((16,), jnp.float32)

        @pl.loop(0, zr)
        def _(r):
            for j in range(8):
                zc_v[r, pl.ds(16 * j, 16)] = zero16

        @pl.loop(0, _CH)
        def _(r):
            for j in range(8):
                ones_v[r, pl.ds(16 * j, 16)] = one16

        base = sid * nps

        @pl.loop(0, nz)
        def _(z):
            pltpu.sync_copy(zc_v, cacc_sh.at[pl.ds(base + z * zr, zr)])

        plsc.subcore_barrier()

        def issue_i(step, bi):
            off = (w + step * _NW) * _CH
            pltpu.async_copy(dst_hbm.at[pl.ds(off, _CH)], dstb.at[bi],
                             sem_i[bi])

        def wait_i(bi):
            pltpu.make_async_copy(dst_hbm.at[pl.ds(0, _CH)], dstb.at[bi],
                                  sem_i[bi]).wait()

        def issue_s(bi, b):
            pltpu.async_copy(ones_v, cacc_sh.at[dstb.at[bi]], sem_s[b],
                             add=True)

        def wait_s(b):
            pltpu.make_async_copy(cnt_hbm.at[pl.ds(0, _CH)], ones_v,
                                  sem_s[b]).wait()

        issue_i(0, 0)
        issue_i(1, 1)
        wait_i(0)
        issue_s(0, 0)
        issue_i(2, 2)
        wait_i(1)
        issue_s(1, 1)
        issue_i(3, 3)

        @pl.loop(0, (steps - 4) // 4)
        def _(o):
            for u in range(4):
                step = 2 + o * 4 + u
                b = u % 2
                bi = (2 + u) % 4
                wait_i(bi)
                issue_s(bi, b)
                wait_s(b)
                issue_i(step + 2, (bi + 2) % 4)

        wait_i(2)
        issue_s(2, 0)
        wait_s(0)
        wait_i(3)
        issue_s(3, 1)
        wait_s(1)
        wait_s(0)
        wait_s(1)

        plsc.subcore_barrier()
        obase = cid * npad + base
        pltpu.sync_copy(cacc_sh.at[pl.ds(base, nps)],
                        cnt_hbm.at[pl.ds(obase, nps)])

    mesh = plsc.VectorSubcoreMesh(core_axis_name="c", subcore_axis_name="s")
    f = pl.kernel(body,
                  out_type=jax.ShapeDtypeStruct((_NC * npad, 128),
                                                jnp.float32),
                  mesh=mesh, scratch_types=scratch)
    return f(dst)


def _sc_pool(h, batch, g):
    n, d = h.shape
    full = n // _CH
    assert full * _CH == n
    iters = -(-full // _NW)
    ga = g + 8
    assert g % 8 == 0 and ga // 8 <= _NS

    scratch = [
        pltpu.VMEM((_CH,), jnp.int32),
        pltpu.VMEM((_CH, d), jnp.float32),
        pltpu.VMEM((_CH, 128), jnp.float32),
        pltpu.VMEM((8, d), jnp.float32),
        pltpu.VMEM((8, 128), jnp.float32),
        pltpu.VMEM_SHARED((ga, d), jnp.float32),
        pltpu.VMEM_SHARED((ga, 128), jnp.float32),
        pltpu.SemaphoreType.DMA,
    ]

    def body(h_hbm, b_hbm, out_hbm, cnt_hbm, bidx_v, rows_v, ones_v,
             zrow_v, zc_v, acc_sh, cacc_sh, sem):
        cid = lax.axis_index("c")
        sid = lax.axis_index("s")
        w = sid * _NC + cid

        zero16 = jnp.zeros((16,), jnp.float32)
        one16 = jnp.ones((16,), jnp.float32)

        @pl.loop(0, 8)
        def _(r):
            for j in range(d // 16):
                zrow_v[r, pl.ds(16 * j, 16)] = zero16
            for j in range(8):
                zc_v[r, pl.ds(16 * j, 16)] = zero16

        @pl.loop(0, _CH)
        def _(r):
            for j in range(8):
                ones_v[r, pl.ds(16 * j, 16)] = one16

        base = sid * 8

        @pl.when(sid < ga // 8)
        def _():
            pltpu.sync_copy(zrow_v, acc_sh.at[pl.ds(base, 8)])
            pltpu.sync_copy(zc_v, cacc_sh.at[pl.ds(base, 8)])

        plsc.subcore_barrier()

        @pl.loop(0, iters)
        def _(i):
            c = w + i * _NW

            @pl.when(c < full)
            def _():
                off = c * _CH
                pltpu.sync_copy(b_hbm.at[pl.ds(off, _CH)], bidx_v)
                pltpu.sync_copy(h_hbm.at[pl.ds(off, _CH)], rows_v)
                pltpu.sync_copy(rows_v, acc_sh.at[bidx_v], add=True)
                pltpu.sync_copy(ones_v, cacc_sh.at[bidx_v], add=True)

        plsc.subcore_barrier()
        obase = cid * g + base

        @pl.when(sid < g // 8)
        def _():
            pltpu.sync_copy(acc_sh.at[pl.ds(base, 8)],
                            out_hbm.at[pl.ds(obase, 8)])
            pltpu.sync_copy(cacc_sh.at[pl.ds(base, 8)],
                            cnt_hbm.at[pl.ds(obase, 8)])

    mesh = plsc.VectorSubcoreMesh(core_axis_name="c", subcore_axis_name="s")
    f = pl.kernel(body,
                  out_type=(jax.ShapeDtypeStruct((_NC * g, d), jnp.float32),
                            jax.ShapeDtypeStruct((_NC * g, 128),
                                                 jnp.float32)),
                  mesh=mesh, scratch_types=scratch)
    return f(h, batch)


def _tc_sage_linear(parts, cnts, x, wl, bias, wr, npad):
    d = x.shape[1]
    br = _largest_div_le(npad, 1024)
    while br % 8 != 0:
        br //= 2
    nb = npad // br
    off = nb

    def body(p0, p1, c0, c1, x_ref, wl_ref, wr_ref, b_ref, o_ref):
        s = p0[...] + p1[...]
        c = c0[...][:, 0:1] + c1[...][:, 0:1]
        agg = s / jnp.maximum(c, 1.0)
        h = (lax.dot_general(agg, wl_ref[...], (((1,), (1,)), ((), ())),
                             preferred_element_type=jnp.float32)
             + lax.dot_general(x_ref[...], wr_ref[...],
                               (((1,), (1,)), ((), ())),
                               preferred_element_type=jnp.float32)
             + b_ref[...])
        o_ref[...] = jnp.maximum(h, 0.0)

    row_spec = pl.BlockSpec((br, d), lambda i: (i, 0))
    return pl.pallas_call(
        body,
        grid=(nb,),
        in_specs=[
            row_spec,
            pl.BlockSpec((br, d), lambda i: (i + off, 0)),
            pl.BlockSpec((br, 128), lambda i: (i, 0)),
            pl.BlockSpec((br, 128), lambda i: (i + off, 0)),
            row_spec,
            pl.BlockSpec((d, d), lambda i: (0, 0)),
            pl.BlockSpec((d, d), lambda i: (0, 0)),
            pl.BlockSpec((1, d), lambda i: (0, 0)),
        ],
        out_specs=row_spec,
        out_shape=jax.ShapeDtypeStruct((npad, d), jnp.float32),
    )(parts, parts, cnts, cnts, x, wl, wr, bias.reshape(1, d))


def _tc_head(pparts, pcnts, wc, bias, g, d):
    def body(p0, p1, c0, c1, w_ref, b_ref, o_ref):
        p = p0[...] + p1[...]
        c = c0[...][:, 0:1] + c1[...][:, 0:1]
        pooled = p / jnp.maximum(c, 1.0)
        o_ref[...] = (lax.dot_general(pooled, w_ref[...],
                                      (((1,), (1,)), ((), ())),
                                      preferred_element_type=jnp.float32)
                      + b_ref[...])

    return pl.pallas_call(
        body,
        grid=(1,),
        in_specs=[
            pl.BlockSpec((g, d), lambda i: (0, 0)),
            pl.BlockSpec((g, d), lambda i: (1, 0)),
            pl.BlockSpec((g, 128), lambda i: (0, 0)),
            pl.BlockSpec((g, 128), lambda i: (1, 0)),
            pl.BlockSpec((d, d), lambda i: (0, 0)),
            pl.BlockSpec((1, d), lambda i: (0, 0)),
        ],
        out_specs=pl.BlockSpec((g, d), lambda i: (0, 0)),
        out_shape=jax.ShapeDtypeStruct((g, d), jnp.float32),
    )(pparts, pparts, pcnts, pcnts, wc, bias.reshape(1, d))


def kernel(x, edge_index, batch, W1l, b1l, W1r, W2l, b2l, W2r, Wc, bc):
    g = 64
    n, d = x.shape
    e = edge_index.shape[1]
    npad, _ = _acc_rows(n)
    assert npad > n

    steps = -(-e // (_NW * _CH))
    steps = max(-(-steps // 4) * 4, 8)
    epad = steps * _NW * _CH
    pad_src = jnp.arange(epad - e, dtype=jnp.int32) % n
    src = jnp.concatenate([edge_index[0], pad_src])
    pad_dst = n + jnp.arange(epad - e, dtype=jnp.int32) % (npad - n)
    dst = jnp.concatenate([edge_index[1], pad_dst])
    parts1, npad1 = _sc_edge_aggregate(x, src, dst)
    cnts = _sc_degree_count(dst, n)
    h1 = _tc_sage_linear(parts1, cnts, x, W1l, b1l, W1r, npad1)
    parts2, npad2 = _sc_edge_aggregate(h1, src, dst)
    h2 = _tc_sage_linear(parts2, cnts, h1, W2l, b2l, W2r, npad2)

    batch_p = jnp.concatenate(
        [batch, jnp.full((h2.shape[0] - n,), g, jnp.int32)])
    pparts, pcnts = _sc_pool(h2, batch_p, g)
    return _tc_head(pparts, pcnts, Wc, bc, g, d)


# speedup vs baseline: 2.9402x; 1.5832x over previous
import jax
import jax.numpy as jnp
from jax import lax
from jax.experimental import pallas as pl
from jax.experimental.pallas import tpu as pltpu
from jax.experimental.pallas import tpu_sc as plsc

_NC = 2
_NS = 16
_NW = _NC * _NS
_CH = 128


def _largest_div_le(n, cap):
    for cand in range(min(n, cap), 0, -1):
        if n % cand == 0:
            return cand
    return 1


def _acc_rows(n):
    nps = -(-n // (_NS * 8)) * 8
    return nps * _NS, nps


def _sc_edge_aggregate(table, src, dst):
    n, d = table.shape
    e = src.shape[0]
    nchunk = e // _CH
    steps = nchunk // _NW
    assert nchunk == steps * _NW and steps % 4 == 0 and steps >= 8
    npad, nps = _acc_rows(n)
    zr = _largest_div_le(nps, 128)
    nz = nps // zr

    scratch = [
        pltpu.VMEM((4, _CH), jnp.int32),
        pltpu.VMEM((4, _CH), jnp.int32),
        pltpu.VMEM((_CH, d), jnp.float32),
        pltpu.VMEM((_CH, d), jnp.float32),
        pltpu.VMEM((zr, d), jnp.float32),
        pltpu.VMEM_SHARED((npad, d), jnp.float32),
    ] + [pltpu.SemaphoreType.DMA] * 6

    def body(x_hbm, src_hbm, dst_hbm, out_hbm, srcb, dstb, rows0, rows1,
             zrow_v, acc_sh, si0, si1, si2, si3, sg0, sg1):
        cid = lax.axis_index("c")
        sid = lax.axis_index("s")
        w = sid * _NC + cid
        sem_i = (si0, si1, si2, si3)
        sem_g = (sg0, sg1)
        rows = (rows0, rows1)

        zero16 = jnp.zeros((16,), jnp.float32)

        @pl.loop(0, zr)
        def _(r):
            for j in range(d // 16):
                zrow_v[r, pl.ds(16 * j, 16)] = zero16

        base = sid * nps

        @pl.loop(0, nz)
        def _(z):
            pltpu.sync_copy(zrow_v, acc_sh.at[pl.ds(base + z * zr, zr)])

        plsc.subcore_barrier()

        def issue_i(step, bi):
            off = (w + step * _NW) * _CH
            pltpu.async_copy(src_hbm.at[pl.ds(off, _CH)], srcb.at[bi],
                             sem_i[bi])
            pltpu.async_copy(dst_hbm.at[pl.ds(off, _CH)], dstb.at[bi],
                             sem_i[bi])

        def wait_i(bi):
            pltpu.make_async_copy(src_hbm.at[pl.ds(0, _CH)], srcb.at[bi],
                                  sem_i[bi]).wait()
            pltpu.make_async_copy(dst_hbm.at[pl.ds(0, _CH)], dstb.at[bi],
                                  sem_i[bi]).wait()

        def issue_g(bi, rb):
            pltpu.async_copy(x_hbm.at[srcb.at[bi]], rows[rb], sem_g[rb])

        def wait_g(rb):
            pltpu.make_async_copy(x_hbm.at[pl.ds(0, _CH)], rows[rb],
                                  sem_g[rb]).wait()

        def sync_s(bi, rb):
            pltpu.sync_copy(rows[rb], acc_sh.at[dstb.at[bi]], add=True)

        issue_i(0, 0)
        issue_i(1, 1)
        wait_i(0)
        issue_g(0, 0)
        wait_g(0)
        wait_i(1)
        issue_g(1, 1)
        sync_s(0, 0)
        issue_i(2, 2)
        wait_g(1)
        wait_i(2)
        issue_g(2, 0)
        sync_s(1, 1)
        issue_i(3, 3)

        @pl.loop(0, (steps - 4) // 4)
        def _(o):
            for u in range(4):
                step = 2 + o * 4 + u
                b = u % 2
                bi = (2 + u) % 4
                wait_g(b)
                wait_i((bi + 1) % 4)
                issue_g((bi + 1) % 4, 1 - b)
                sync_s(bi, b)
                issue_i(step + 2, (bi + 2) % 4)

        wait_g(0)
        wait_i(3)
        issue_g(3, 1)
        sync_s(2, 0)
        wait_g(1)
        sync_s(3, 1)

        plsc.subcore_barrier()
        obase = cid * npad + base
        pltpu.sync_copy(acc_sh.at[pl.ds(base, nps)],
                        out_hbm.at[pl.ds(obase, nps)])

    mesh = plsc.VectorSubcoreMesh(core_axis_name="c", subcore_axis_name="s")
    f = pl.kernel(body,
                  out_type=jax.ShapeDtypeStruct((_NC * npad, d), jnp.float32),
                  mesh=mesh, scratch_types=scratch)
    return f(table, src, dst), npad


def _sc_degree_count(dst, n):
    e = dst.shape[0]
    nchunk = e // _CH
    steps = nchunk // _NW
    assert nchunk == steps * _NW and steps % 4 == 0 and steps >= 8
    npad, nps = _acc_rows(n)
    zr = _largest_div_le(nps, 128)
    nz = nps // zr

    scratch = [
        pltpu.VMEM((4, _CH), jnp.int32),
        pltpu.VMEM((zr, 128), jnp.float32),
        pltpu.VMEM((_CH, 128), jnp.float32),
        pltpu.VMEM_SHARED((npad, 128), jnp.float32),
    ] + [pltpu.SemaphoreType.DMA] * 6

    def body(dst_hbm, cnt_hbm, dstb, zc_v, ones_v, cacc_sh,
             si0, si1, si2, si3, ss0, ss1):
        cid = lax.axis_index("c")
        sid = lax.axis_index("s")
        w = sid * _NC + cid
        sem_i = (si0, si1, si2, si3)
        sem_s = (ss0, ss1)

        zero16 = jnp.zeros((16,), jnp.float32)
        one16 = jnp.ones((16,), jnp.float32)

        @pl.loop(0, zr)
        def _(r):
            for j in range(8):
                zc_v[r, pl.ds(16 * j, 16)] = zero16

        @pl.loop(0, _CH)
        def _(r):
            for j in range(8):
                ones_v[r, pl.ds(16 * j, 16)] = one16

        base = sid * nps

        @pl.loop(0, nz)
        def _(z):
            pltpu.sync_copy(zc_v, cacc_sh.at[pl.ds(base + z * zr, zr)])

        plsc.subcore_barrier()

        def issue_i(step, bi):
            off = (w + step * _NW) * _CH
            pltpu.async_copy(dst_hbm.at[pl.ds(off, _CH)], dstb.at[bi],
                             sem_i[bi])

        def wait_i(bi):
            pltpu.make_async_copy(dst_hbm.at[pl.ds(0, _CH)], dstb.at[bi],
                                  sem_i[bi]).wait()

        def issue_s(bi, b):
            pltpu.async_copy(ones_v, cacc_sh.at[dstb.at[bi]], sem_s[b],
                             add=True)

        def wait_s(b):
            pltpu.make_async_copy(cnt_hbm.at[pl.ds(0, _CH)], ones_v,
                                  sem_s[b]).wait()

        issue_i(0, 0)
        issue_i(1, 1)
        wait_i(0)
        issue_s(0, 0)
        issue_i(2, 2)
        wait_i(1)
        issue_s(1, 1)
        issue_i(3, 3)

        @pl.loop(0, (steps - 4) // 4)
        def _(o):
            for u in range(4):
                step = 2 + o * 4 + u
                b = u % 2
                bi = (2 + u) % 4
                wait_i(bi)
                issue_s(bi, b)
                wait_s(b)
                issue_i(step + 2, (bi + 2) % 4)

        wait_i(2)
        issue_s(2, 0)
        wait_s(0)
        wait_i(3)
        issue_s(3, 1)
        wait_s(1)
        wait_s(0)
        wait_s(1)

        plsc.subcore_barrier()
        obase = cid * npad + base
        pltpu.sync_copy(cacc_sh.at[pl.ds(base, nps)],
                        cnt_hbm.at[pl.ds(obase, nps)])

    mesh = plsc.VectorSubcoreMesh(core_axis_name="c", subcore_axis_name="s")
    f = pl.kernel(body,
                  out_type=jax.ShapeDtypeStruct((_NC * npad, 128),
                                                jnp.float32),
                  mesh=mesh, scratch_types=scratch)
    return f(dst)


def _sc_pool(h, batch, g):
    n, d = h.shape
    full = n // _CH
    assert full * _CH == n
    iters = -(-full // _NW)
    ga = g + 8
    assert g % 8 == 0 and ga // 8 <= _NS

    scratch = [
        pltpu.VMEM((_CH,), jnp.int32),
        pltpu.VMEM((_CH, d), jnp.float32),
        pltpu.VMEM((_CH, 128), jnp.float32),
        pltpu.VMEM((8, d), jnp.float32),
        pltpu.VMEM((8, 128), jnp.float32),
        pltpu.VMEM_SHARED((ga, d), jnp.float32),
        pltpu.VMEM_SHARED((ga, 128), jnp.float32),
        pltpu.SemaphoreType.DMA,
    ]

    def body(h_hbm, b_hbm, out_hbm, cnt_hbm, bidx_v, rows_v, ones_v,
             zrow_v, zc_v, acc_sh, cacc_sh, sem):
        cid = lax.axis_index("c")
        sid = lax.axis_index("s")
        w = sid * _NC + cid

        zero16 = jnp.zeros((16,), jnp.float32)
        one16 = jnp.ones((16,), jnp.float32)

        @pl.loop(0, 8)
        def _(r):
            for j in range(d // 16):
                zrow_v[r, pl.ds(16 * j, 16)] = zero16
            for j in range(8):
                zc_v[r, pl.ds(16 * j, 16)] = zero16

        @pl.loop(0, _CH)
        def _(r):
            for j in range(8):
                ones_v[r, pl.ds(16 * j, 16)] = one16

        base = sid * 8

        @pl.when(sid < ga // 8)
        def _():
            pltpu.sync_copy(zrow_v, acc_sh.at[pl.ds(base, 8)])
            pltpu.sync_copy(zc_v, cacc_sh.at[pl.ds(base, 8)])

        plsc.subcore_barrier()

        @pl.loop(0, iters)
        def _(i):
            c = w + i * _NW

            @pl.when(c < full)
            def _():
                off = c * _CH
                pltpu.sync_copy(b_hbm.at[pl.ds(off, _CH)], bidx_v)
                pltpu.sync_copy(h_hbm.at[pl.ds(off, _CH)], rows_v)
                pltpu.sync_copy(rows_v, acc_sh.at[bidx_v], add=True)
                pltpu.sync_copy(ones_v, cacc_sh.at[bidx_v], add=True)

        plsc.subcore_barrier()
        obase = cid * g + base

        @pl.when(sid < g // 8)
        def _():
            pltpu.sync_copy(acc_sh.at[pl.ds(base, 8)],
                            out_hbm.at[pl.ds(obase, 8)])
            pltpu.sync_copy(cacc_sh.at[pl.ds(base, 8)],
                            cnt_hbm.at[pl.ds(obase, 8)])

    mesh = plsc.VectorSubcoreMesh(core_axis_name="c", subcore_axis_name="s")
    f = pl.kernel(body,
                  out_type=(jax.ShapeDtypeStruct((_NC * g, d), jnp.float32),
                            jax.ShapeDtypeStruct((_NC * g, 128),
                                                 jnp.float32)),
                  mesh=mesh, scratch_types=scratch)
    return f(h, batch)


def _tc_sage_linear(parts, cnts, x, wl, bias, wr, npad):
    d = x.shape[1]
    br = _largest_div_le(npad, 1024)
    while br % 8 != 0:
        br //= 2
    nb = npad // br
    off = nb

    def body(p0, p1, c0, c1, x_ref, wl_ref, wr_ref, b_ref, o_ref):
        s = p0[...] + p1[...]
        c = c0[...][:, 0:1] + c1[...][:, 0:1]
        agg = s / jnp.maximum(c, 1.0)
        h = (lax.dot_general(agg, wl_ref[...], (((1,), (1,)), ((), ())),
                             preferred_element_type=jnp.float32)
             + lax.dot_general(x_ref[...], wr_ref[...],
                               (((1,), (1,)), ((), ())),
                               preferred_element_type=jnp.float32)
             + b_ref[...])
        o_ref[...] = jnp.maximum(h, 0.0)

    row_spec = pl.BlockSpec((br, d), lambda i: (i, 0))
    return pl.pallas_call(
        body,
        grid=(nb,),
        in_specs=[
            row_spec,
            pl.BlockSpec((br, d), lambda i: (i + off, 0)),
            pl.BlockSpec((br, 128), lambda i: (i, 0)),
            pl.BlockSpec((br, 128), lambda i: (i + off, 0)),
            row_spec,
            pl.BlockSpec((d, d), lambda i: (0, 0)),
            pl.BlockSpec((d, d), lambda i: (0, 0)),
            pl.BlockSpec((1, d), lambda i: (0, 0)),
        ],
        out_specs=row_spec,
        out_shape=jax.ShapeDtypeStruct((npad, d), jnp.float32),
    )(parts, parts, cnts, cnts, x, wl, wr, bias.reshape(1, d))


def _tc_head(pparts, pcnts, wc, bias, g, d):
    def body(p0, p1, c0, c1, w_ref, b_ref, o_ref):
        p = p0[...] + p1[...]
        c = c0[...][:, 0:1] + c1[...][:, 0:1]
        pooled = p / jnp.maximum(c, 1.0)
        o_ref[...] = (lax.dot_general(pooled, w_ref[...],
                                      (((1,), (1,)), ((), ())),
                                      preferred_element_type=jnp.float32)
                      + b_ref[...])

    return pl.pallas_call(
        body,
        grid=(1,),
        in_specs=[
            pl.BlockSpec((g, d), lambda i: (0, 0)),
            pl.BlockSpec((g, d), lambda i: (1, 0)),
            pl.BlockSpec((g, 128), lambda i: (0, 0)),
            pl.BlockSpec((g, 128), lambda i: (1, 0)),
            pl.BlockSpec((d, d), lambda i: (0, 0)),
            pl.BlockSpec((1, d), lambda i: (0, 0)),
        ],
        out_specs=pl.BlockSpec((g, d), lambda i: (0, 0)),
        out_shape=jax.ShapeDtypeStruct((g, d), jnp.float32),
    )(pparts, pparts, pcnts, pcnts, wc, bias.reshape(1, d))


def kernel(x, edge_index, batch, W1l, b1l, W1r, W2l, b2l, W2r, Wc, bc):
    g = 64
    n, d = x.shape
    e = edge_index.shape[1]
    npad, _ = _acc_rows(n)
    assert npad > n

    steps = -(-e // (_NW * _CH))
    steps = max(-(-steps // 4) * 4, 8)
    epad = steps * _NW * _CH
    pad_src = jnp.arange(epad - e, dtype=jnp.int32) % n
    src = jnp.concatenate([edge_index[0], pad_src])
    pad_dst = n + jnp.arange(epad - e, dtype=jnp.int32) % (npad - n)
    dst = jnp.concatenate([edge_index[1], pad_dst])
    parts1, npad1 = _sc_edge_aggregate(x, src, dst)
    cnts = _sc_degree_count(dst, n)
    h1 = _tc_sage_linear(parts1, cnts, x, W1l, b1l, W1r, npad1)
    parts2, npad2 = _sc_edge_aggregate(h1, src, dst)
    h2 = _tc_sage_linear(parts2, cnts, h1, W2l, b2l, W2r, npad2)

    batch_p = jnp.concatenate(
        [batch, jnp.full((h2.shape[0] - n,), g, jnp.int32)])
    pparts, pcnts = _sc_pool(h2, batch_p, g)
    return _tc_head(pparts, pcnts, Wc, bc, g, d)
